# Initial kernel scaffold; baseline (speedup 1.0000x reference)
import jax
import jax.numpy as jnp
from jax.experimental import pallas as pl


def kernel(x_ab, x_ag, pe_ab, pe_ag, y_abab, y_agag, y_abag, y_agab, ei_abab, ei_agag, ei_abag, ei_agab, Wq_ca, Wk_ca, Wv_ca, Wo_ca, We_int, be_int, We_all, be_all, Wq_gt, Wk_gt, Wv_gt, Wo_gt, wb_gt, W1_ff, W2_ff, We_fin, be_fin):
    raise NotImplementedError("write your pallas kernel here")


# speedup vs baseline: 6.9061x; 6.9061x over previous
import functools
import math

import jax
import jax.numpy as jnp
from jax import lax
from jax.experimental import pallas as pl
from jax.experimental.pallas import tpu as pltpu
from jax.experimental.pallas import tpu_sc as plsc

N_AB = 5000
N_AG = 5000
E = 40000
D = 256
H = 8
DH = 32
BLOCKS = 3
DFF = 512

NP = 5120
NNP = 2 * NP
EP = 40960
EG = 4 * EP

NC = 2
NS = 16
NW = NC * NS
CH = 128

BM = 256
F32 = jnp.float32
SCALE = 1.0 / math.sqrt(DH)


def _sc_mesh():
    return plsc.VectorSubcoreMesh(
        core_axis_name="c", subcore_axis_name="s", num_cores=NC, num_subcores=NS
    )


@functools.cache
def _gather_kernel(n_rows, d, ep):
    cpw = ep // NW // CH

    @functools.partial(
        pl.kernel,
        out_type=jax.ShapeDtypeStruct((ep, d), F32),
        mesh=_sc_mesh(),
        scratch_types=[
            pltpu.VMEM((CH,), jnp.int32),
            pltpu.VMEM((CH, d), F32),
            pltpu.SemaphoreType.DMA,
        ],
    )
    def k(table, idx2d, out, idx_v, rows_v, sem):
        wid = lax.axis_index("s") * NC + lax.axis_index("c")

        def body(i, carry):
            c0 = wid * cpw + i
            pltpu.sync_copy(idx2d.at[c0], idx_v)
            pltpu.async_copy(table.at[idx_v], rows_v, sem).wait()
            pltpu.sync_copy(rows_v, out.at[pl.ds(c0 * CH, CH)])
            return carry

        lax.fori_loop(0, cpw, body, 0)

    return k


def _gather(table, idx2d):
    n_rows, d = table.shape
    ep = idx2d.shape[0] * CH
    return _gather_kernel(n_rows, d, ep)(table, idx2d)


@functools.cache
def _scatter_kernel(n_pad, dcol, ep):
    cpw = ep // NW // CH
    rps = n_pad // NS

    @functools.partial(
        pl.kernel,
        out_type=jax.ShapeDtypeStruct((NC, n_pad, dcol), F32),
        mesh=_sc_mesh(),
        scratch_types=[
            pltpu.VMEM((1, CH), jnp.int32),
            pltpu.VMEM((CH, dcol), F32),
            pltpu.VMEM_SHARED((n_pad, dcol), F32),
            pltpu.SemaphoreType.DMA,
        ],
    )
    def k(msg, idx2d, zeros, out, idx_v, msg_v, shared, sem):
        cid = lax.axis_index("c")
        sid = lax.axis_index("s")
        wid = sid * NC + cid

        pltpu.sync_copy(
            zeros.at[pl.ds(sid * rps, rps)], shared.at[pl.ds(sid * rps, rps)]
        )
        plsc.subcore_barrier()

        def body(i, carry):
            c0 = wid * cpw + i
            pltpu.sync_copy(idx2d.at[pl.ds(c0, 1)], idx_v)
            pltpu.sync_copy(msg.at[pl.ds(c0 * CH, CH)], msg_v)
            pltpu.sync_copy(msg_v, shared.at[idx_v.at[0]], add=True)
            return carry

        lax.fori_loop(0, cpw, body, 0)
        plsc.subcore_barrier()
        pltpu.sync_copy(
            shared.at[pl.ds(sid * rps, rps)],
            out.at[cid, pl.ds(sid * rps, rps)],
        )

    return k


def _scatter_add(msg, idx2d, n_pad):
    ep, dcol = msg.shape
    zeros = jnp.zeros((n_pad, dcol), F32)
    return _scatter_kernel(n_pad, dcol, ep)(msg, idx2d, zeros)


def _dot(a, b):
    return jnp.dot(a, b, preferred_element_type=F32)


def _sel_dn(rows, cols, off):
    hi = lax.broadcasted_iota(jnp.int32, (rows, cols), 0)
    ci = lax.broadcasted_iota(jnp.int32, (rows, cols), 1)
    return (hi == off + ci // DH).astype(F32)


def _sel_up(rows, cols):
    di = lax.broadcasted_iota(jnp.int32, (rows, cols), 0)
    hi = lax.broadcasted_iota(jnp.int32, (rows, cols), 1)
    return (di // DH == hi).astype(F32)


def _mm_body(x_ref, w_ref, o_ref):
    o_ref[...] = _dot(x_ref[...], w_ref[...])


@functools.cache
def _mm_kernel(m, kdim, n):
    return pl.pallas_call(
        _mm_body,
        grid=(m // BM,),
        in_specs=[
            pl.BlockSpec((BM, kdim), lambda i: (i, 0)),
            pl.BlockSpec((kdim, n), lambda i: (0, 0)),
        ],
        out_specs=pl.BlockSpec((BM, n), lambda i: (i, 0)),
        out_shape=jax.ShapeDtypeStruct((m, n), F32),
    )


def _mm(x, w):
    return _mm_kernel(x.shape[0], x.shape[1], w.shape[1])(x, w)


def _addpe_body(x_ref, p_ref, o_ref):
    o_ref[...] = x_ref[...] + p_ref[...]


@functools.cache
def _addpe_kernel(m):
    return pl.pallas_call(
        _addpe_body,
        grid=(m // BM,),
        in_specs=[
            pl.BlockSpec((BM, D), lambda i: (i, 0)),
            pl.BlockSpec((BM, D), lambda i: (i, 0)),
        ],
        out_specs=pl.BlockSpec((BM, D), lambda i: (i, 0)),
        out_shape=jax.ShapeDtypeStruct((m, D), F32),
    )


def _score_body(with_bias, qd_ref, kvs_ref, *rest):
    if with_bias:
        y_ref, wb_ref, sc_ref, gm_ref = rest
    else:
        sc_ref, gm_ref = rest
    i = pl.program_id(0)
    prod = qd_ref[...] * kvs_ref[:, :D]
    sc = _dot(prod, _sel_up(D, H)) * SCALE
    if with_bias:
        sc = sc + _dot(y_ref[...], wb_ref[...])
    sc_ref[...] = sc
    bmax = jnp.full((8, 128), jnp.max(sc), F32)

    @pl.when(i == 0)
    def _():
        gm_ref[...] = bmax

    @pl.when(i > 0)
    def _():
        gm_ref[...] = jnp.maximum(gm_ref[...], bmax)


@functools.cache
def _score_kernel(ep, with_bias):
    in_specs = [
        pl.BlockSpec((BM, D), lambda i: (i, 0)),
        pl.BlockSpec((BM, 2 * D), lambda i: (i, 0)),
    ]
    if with_bias:
        in_specs += [
            pl.BlockSpec((BM, D), lambda i: (i, 0)),
            pl.BlockSpec((D, H), lambda i: (0, 0)),
        ]
    return pl.pallas_call(
        functools.partial(_score_body, with_bias),
        grid=(ep // BM,),
        in_specs=in_specs,
        out_specs=[
            pl.BlockSpec((BM, H), lambda i: (i, 0)),
            pl.BlockSpec((8, 128), lambda i: (0, 0)),
        ],
        out_shape=[
            jax.ShapeDtypeStruct((ep, H), F32),
            jax.ShapeDtypeStruct((8, 128), F32),
        ],
    )


def _msg_body(period, valid, sc_ref, kvs_ref, gm_ref, ml_ref, mh_ref, ex_ref):
    i = pl.program_id(0)
    rows = i * BM + lax.broadcasted_iota(jnp.int32, (BM, H), 0)
    mask = (lax.rem(rows, period) < valid).astype(F32)
    g = gm_ref[...][0:1, 0:1]
    ex = jnp.exp(sc_ref[...] - g) * mask
    vs = kvs_ref[:, D:]
    ml_ref[...] = _dot(ex, _sel_dn(H, 128, 0)) * vs[:, :128]
    mh_ref[...] = _dot(ex, _sel_dn(H, 128, 4)) * vs[:, 128:]
    pi = lax.broadcasted_iota(jnp.int32, (H, 128), 0)
    pj = lax.broadcasted_iota(jnp.int32, (H, 128), 1)
    ex_ref[...] = _dot(ex, (pi == pj).astype(F32))


@functools.cache
def _msg_kernel(ep):
    return pl.pallas_call(
        functools.partial(_msg_body, EP, E),
        grid=(ep // BM,),
        in_specs=[
            pl.BlockSpec((BM, H), lambda i: (i, 0)),
            pl.BlockSpec((BM, 2 * D), lambda i: (i, 0)),
            pl.BlockSpec((8, 128), lambda i: (0, 0)),
        ],
        out_specs=[
            pl.BlockSpec((BM, 128), lambda i: (i, 0)),
            pl.BlockSpec((BM, 128), lambda i: (i, 0)),
            pl.BlockSpec((BM, 128), lambda i: (i, 0)),
        ],
        out_shape=[
            jax.ShapeDtypeStruct((ep, 128), F32),
            jax.ShapeDtypeStruct((ep, 128), F32),
            jax.ShapeDtypeStruct((ep, 128), F32),
        ],
    )


def _edgenet_body(res, xs_ref, xd_ref, y_ref, w_ref, b_ref, o_ref):
    acc = (
        _dot(xs_ref[...], w_ref[:D, :])
        + _dot(xd_ref[...], w_ref[D : 2 * D, :])
        + _dot(y_ref[...], w_ref[2 * D :, :])
        + b_ref[...][0:1, :]
    )
    out = jnp.maximum(acc, 0.0)
    if res:
        out = out + y_ref[...]
    o_ref[...] = out


@functools.cache
def _edgenet_kernel(ep, res):
    return pl.pallas_call(
        functools.partial(_edgenet_body, res),
        grid=(ep // BM,),
        in_specs=[
            pl.BlockSpec((BM, D), lambda i: (i, 0)),
            pl.BlockSpec((BM, D), lambda i: (i, 0)),
            pl.BlockSpec((BM, D), lambda i: (i, 0)),
            pl.BlockSpec((3 * D, D), lambda i: (0, 0)),
            pl.BlockSpec((8, D), lambda i: (0, 0)),
        ],
        out_specs=pl.BlockSpec((BM, D), lambda i: (i, 0)),
        out_shape=jax.ShapeDtypeStruct((ep, D), F32),
    )


def _edgenet(xs_g, xd_g, y, w, b8, res):
    return _edgenet_kernel(y.shape[0], res)(xs_g, xd_g, y, w, b8)


def _combine_body(with_ff, a0l_ref, a1l_ref, a0h_ref, a1h_ref, s0_ref, s1_ref,
                  x_ref, wo_ref, *rest):
    if with_ff:
        w1_ref, w2_ref, o_ref = rest
    else:
        (o_ref,) = rest
    s = s0_ref[...] + s1_ref[...]
    dl = _dot(s, _sel_dn(128, 128, 0))
    dh = _dot(s, _sel_dn(128, 128, 4))
    numl = a0l_ref[...] + a1l_ref[...]
    numh = a0h_ref[...] + a1h_ref[...]
    aggl = jnp.where(dl > 0.0, numl / jnp.where(dl > 0.0, dl, 1.0), 0.0)
    aggh = jnp.where(dh > 0.0, numh / jnp.where(dh > 0.0, dh, 1.0), 0.0)
    h = x_ref[...] + _dot(aggl, wo_ref[:128, :]) + _dot(aggh, wo_ref[128:, :])
    if with_ff:
        h = h + _dot(jnp.maximum(_dot(h, w1_ref[...]), 0.0), w2_ref[...])
    o_ref[...] = h


@functools.cache
def _combine_kernel(n_pad, with_ff):
    in_specs = [
        pl.BlockSpec((BM, 128), lambda i: (i, 0)),
        pl.BlockSpec((BM, 128), lambda i: (i, 0)),
        pl.BlockSpec((BM, 128), lambda i: (i, 0)),
        pl.BlockSpec((BM, 128), lambda i: (i, 0)),
        pl.BlockSpec((BM, 128), lambda i: (i, 0)),
        pl.BlockSpec((BM, 128), lambda i: (i, 0)),
        pl.BlockSpec((BM, D), lambda i: (i, 0)),
        pl.BlockSpec((D, D), lambda i: (0, 0)),
    ]
    if with_ff:
        in_specs += [
            pl.BlockSpec((D, DFF), lambda i: (0, 0)),
            pl.BlockSpec((DFF, D), lambda i: (0, 0)),
        ]
    return pl.pallas_call(
        functools.partial(_combine_body, with_ff),
        grid=(n_pad // BM,),
        in_specs=in_specs,
        out_specs=pl.BlockSpec((BM, D), lambda i: (i, 0)),
        out_shape=jax.ShapeDtypeStruct((n_pad, D), F32),
    )


def _attention(x_dst, x_src, src2d, dst2d, wq, wkv, wo, n_pad, ep,
               y_all=None, wb=None, w1=None, w2=None):
    q = _mm(x_dst, wq)
    kv = _mm(x_src, wkv)
    qd = _gather(q, dst2d)
    kvs = _gather(kv, src2d)
    if wb is not None:
        sc, gm = _score_kernel(ep, True)(qd, kvs, y_all, wb)
    else:
        sc, gm = _score_kernel(ep, False)(qd, kvs)
    ml, mh, ex = _msg_kernel(ep)(sc, kvs, gm)
    pl_ = _scatter_add(ml, dst2d, n_pad)
    ph_ = _scatter_add(mh, dst2d, n_pad)
    ps_ = _scatter_add(ex, dst2d, n_pad)
    if w1 is not None:
        return _combine_kernel(n_pad, True)(
            pl_[0], pl_[1], ph_[0], ph_[1], ps_[0], ps_[1], x_dst, wo, w1, w2
        )
    return _combine_kernel(n_pad, False)(
        pl_[0], pl_[1], ph_[0], ph_[1], ps_[0], ps_[1], x_dst, wo
    )


def _pad_rows(a, n):
    return jnp.pad(a.astype(F32), ((0, n - a.shape[0]), (0, 0)))


def _idx2d(idx, ep, off=0):
    p = jnp.pad(idx.astype(jnp.int32), (0, ep - idx.shape[0])) + off
    return p.reshape(ep // CH, CH)


def kernel(x_ab, x_ag, pe_ab, pe_ag, y_abab, y_agag, y_abag, y_agab,
           ei_abab, ei_agag, ei_abag, ei_agab,
           Wq_ca, Wk_ca, Wv_ca, Wo_ca, We_int, be_int, We_all, be_all,
           Wq_gt, Wk_gt, Wv_gt, Wo_gt, wb_gt, W1_ff, W2_ff, We_fin, be_fin):
    x_ab = _pad_rows(x_ab, NP)
    x_ag = _pad_rows(x_ag, NP)
    pe_ab = _pad_rows(pe_ab, NP)
    pe_ag = _pad_rows(pe_ag, NP)
    y_abab = _pad_rows(y_abab, EP)
    y_agag = _pad_rows(y_agag, EP)
    y_abag = _pad_rows(y_abag, EP)
    y_agab = _pad_rows(y_agab, EP)

    s_abab, d_abab = _idx2d(ei_abab[0], EP), _idx2d(ei_abab[1], EP)
    s_agag, d_agag = _idx2d(ei_agag[0], EP), _idx2d(ei_agag[1], EP)
    s_abag, d_abag = _idx2d(ei_abag[0], EP), _idx2d(ei_abag[1], EP)
    s_agab, d_agab = _idx2d(ei_agab[0], EP), _idx2d(ei_agab[1], EP)

    g_src = jnp.concatenate([
        _idx2d(ei_abab[0], EP), _idx2d(ei_abag[0], EP),
        _idx2d(ei_agab[0], EP, NP), _idx2d(ei_agag[0], EP, NP),
    ])
    g_dst = jnp.concatenate([
        _idx2d(ei_abab[1], EP), _idx2d(ei_abag[1], EP, NP),
        _idx2d(ei_agab[1], EP), _idx2d(ei_agag[1], EP, NP),
    ])

    def b8(b):
        return jnp.broadcast_to(b.reshape(1, D), (8, D))

    for j in range(BLOCKS):
        x_ab = _addpe_kernel(NP)(x_ab, pe_ab)
        x_ag = _addpe_kernel(NP)(x_ag, pe_ag)

        wkv_ca = jnp.concatenate([Wk_ca[j], Wv_ca[j]], axis=1)
        x_ab = _attention(x_ab, x_ag, s_agab, d_agab,
                          Wq_ca[j], wkv_ca, Wo_ca[j], NP, EP)
        x_ag = _attention(x_ag, x_ab, s_abag, d_abag,
                          Wq_ca[j], wkv_ca, Wo_ca[j], NP, EP)

        g_ab_s_abag = _gather(x_ab, s_abag)
        g_ag_d_abag = _gather(x_ag, d_abag)
        g_ag_s_agab = _gather(x_ag, s_agab)
        g_ab_d_agab = _gather(x_ab, d_agab)
        g_ab_s_abab = _gather(x_ab, s_abab)
        g_ab_d_abab = _gather(x_ab, d_abab)
        g_ag_s_agag = _gather(x_ag, s_agag)
        g_ag_d_agag = _gather(x_ag, d_agag)

        bi, ba, bf = b8(be_int[j]), b8(be_all[j]), b8(be_fin[j])
        y_abag = _edgenet(g_ab_s_abag, g_ag_d_abag, y_abag, We_int[j], bi, j > 0)
        y_agab = _edgenet(g_ag_s_agab, g_ab_d_agab, y_agab, We_int[j], bi, j > 0)
        y_abab = _edgenet(g_ab_s_abab, g_ab_d_abab, y_abab, We_all[j], ba, True)
        y_abag = _edgenet(g_ab_s_abag, g_ag_d_abag, y_abag, We_all[j], ba, True)
        y_agab = _edgenet(g_ag_s_agab, g_ab_d_agab, y_agab, We_all[j], ba, True)
        y_agag = _edgenet(g_ag_s_agag, g_ag_d_agag, y_agag, We_all[j], ba, True)

        xcat = jnp.concatenate([x_ab, x_ag], axis=0)
        y_all = jnp.concatenate([y_abab, y_abag, y_agab, y_agag], axis=0)
        wkv_gt = jnp.concatenate([Wk_gt[j], Wv_gt[j]], axis=1)
        xcat = _attention(xcat, xcat, g_src, g_dst,
                          Wq_gt[j], wkv_gt, Wo_gt[j], NNP, EG,
                          y_all=y_all, wb=wb_gt[j],
                          w1=W1_ff[j], w2=W2_ff[j])
        x_ab, x_ag = xcat[:NP], xcat[NP:]

        f_ab_s_abag = _gather(x_ab, s_abag)
        f_ag_d_abag = _gather(x_ag, d_abag)
        f_ag_s_agab = _gather(x_ag, s_agab)
        f_ab_d_agab = _gather(x_ab, d_agab)
        y_abag = _edgenet(f_ab_s_abag, f_ag_d_abag, y_abag, We_fin[j], bf, False)
        y_agab = _edgenet(f_ag_s_agab, f_ab_d_agab, y_agab, We_fin[j], bf, False)

    return jnp.concatenate([x_ab[:N_AB], x_ag[:N_AG]], axis=0)
